# baseline (speedup 1.0000x reference)
import numpy as np
import jax
import jax.numpy as jnp
from jax import lax
from jax.experimental import pallas as pl
from jax.experimental.pallas import tpu as pltpu
from jax.experimental.pallas import tpu_sc as plsc

_F = 26
_D = 128
_NC = 2
_NS = 16
_NW = _NC * _NS
_C = 16
_G = _C * _F
_P = 208


_DNUMS = lax.GatherDimensionNumbers(
    offset_dims=(), collapsed_slice_dims=(0,), start_index_map=(0,))


def _perm(t, idx):
    return lax.gather(t, idx[:, None], _DNUMS, (1,),
                      mode=lax.GatherScatterMode.PROMISE_IN_BOUNDS)


def _lane_allsum(t):
    iota = lax.broadcasted_iota(jnp.int32, (16,), 0)
    for sh in (8, 4, 2, 1):
        t = t + _perm(t, iota ^ sh)
    return t


def _rsqrt_nr(x):
    i = lax.bitcast_convert_type(x, jnp.int32)
    y = lax.bitcast_convert_type(jnp.int32(0x5F3759DF) - (i >> 1), jnp.float32)
    for _ in range(3):
        y = y * (1.5 - 0.5 * x * y * y)
    return y


def _build(B, L):
    R = B * L
    rows_per_w = R // _NW
    iters = rows_per_w // _C
    words_x = rows_per_w * _F
    mesh = plsc.VectorSubcoreMesh(core_axis_name="c", subcore_axis_name="s")

    def body(x_ref, pat_ref, tbl_ref, gam_ref, bet_ref, out_ref,
             x_v, pat_v, idx0_v, idx1_v, rows_v, out_v, gam_v, bet_v,
             gsem0, gsem1, osem0, osem1):
        wid = lax.axis_index("s") * _NC + lax.axis_index("c")
        base_row = wid * rows_per_w
        pltpu.sync_copy(x_ref.at[pl.ds(base_row * _F, words_x)], x_v)
        pltpu.sync_copy(pat_ref, pat_v)
        pltpu.sync_copy(gam_ref, gam_v)
        pltpu.sync_copy(bet_ref, bet_v)

        gsems = (gsem0, gsem1)
        osems = (osem0, osem1)
        idxs = (idx0_v, idx1_v)


        def compute_idx(it, b):
            off = it * _G
            for j in range(_G // 16):
                idxs[b][pl.ds(j * 16, 16)] = (
                    x_v[pl.ds(off + j * 16, 16)]
                    + pat_v[pl.ds((j % (_P // 16)) * 16, 16)])

        def fire_gather(b):
            for q in range(_G // 104):
                pltpu.async_copy(
                    tbl_ref.at[idxs[b].at[pl.ds(q * 104, 104)]],
                    rows_v.at[b, pl.ds(q * 104, 104)], gsems[b])

        def wait_gather(b):
            pltpu.make_async_copy(
                tbl_ref.at[pl.ds(0, _G)], rows_v.at[b], gsems[b]).wait()

        def fire_out(it, b):
            for q in range(_C // 4):
                r0 = it * _C + q * 4
                bid = wid * (rows_per_w // L) + r0 // L
                l0 = r0 - (r0 // L) * L
                pltpu.async_copy(out_v.at[b, pl.ds(q * 4, 4)],
                                 out_ref.at[bid, pl.ds(l0, 4)], osems[b])

        def wait_out(b):
            pltpu.make_async_copy(
                out_ref.at[0, pl.ds(0, _C - 4)],
                out_v.at[b, pl.ds(0, _C - 4)], osems[b]).wait()
            pltpu.make_async_copy(
                out_ref.at[0, pl.ds(0, 4)],
                out_v.at[b, pl.ds(0, 4)], osems[b]).wait()

        compute_idx(0, 0)
        fire_gather(0)

        def step(i2, carry):
            for b in range(2):
                it = i2 * 2 + b
                nit = it + 1

                @pl.when(nit < iters)
                def _():
                    compute_idx(nit, 1 - b)
                    fire_gather(1 - b)

                wait_gather(b)

                @pl.when(it >= 2)
                def _():
                    wait_out(b)

                def row_body(r, c):
                    rbase = r * _F
                    sh16 = jnp.full((16,), 16, jnp.int32)
                    hi_mask = jnp.full((16,), -65536, jnp.int32)
                    acc = [None] * 8
                    for j in range(_F):
                        for k in range(4):
                            w = rows_v[b, rbase + j, pl.ds(k * 16, 16)]
                            ev = lax.bitcast_convert_type(
                                lax.shift_left(w, sh16), jnp.float32)
                            od = lax.bitcast_convert_type(
                                lax.bitwise_and(w, hi_mask), jnp.float32)
                            if j == 0:
                                acc[k] = ev
                                acc[4 + k] = od
                            else:
                                acc[k] = acc[k] + ev
                                acc[4 + k] = acc[4 + k] + od
                    t = acc[0]
                    for k in range(1, 8):
                        t = t + acc[k]
                    mv = _lane_allsum(t) * (1.0 / _D)
                    d = [a - mv for a in acc]
                    sq = d[0] * d[0]
                    for k in range(1, 8):
                        sq = sq + d[k] * d[k]
                    var = _lane_allsum(sq) * (1.0 / _D)
                    inv = _rsqrt_nr(var + 1e-5)
                    for k in range(8):
                        g = gam_v[pl.ds(k * 16, 16)] * inv
                        out_v[b, r, pl.ds(k * 16, 16)] = (
                            d[k] * g + bet_v[pl.ds(k * 16, 16)])
                    return c

                lax.fori_loop(0, _C, row_body, 0)
                fire_out(it, b)
            return carry

        lax.fori_loop(0, iters // 2, step, 0)
        wait_out(0)
        wait_out(1)

    return pl.kernel(
        body,
        out_type=jax.ShapeDtypeStruct((B, 24, _D), jnp.float32),
        mesh=mesh,
        compiler_params=pltpu.CompilerParams(use_tc_tiling_on_sc=False),
        scratch_types=[
            pltpu.VMEM((words_x,), jnp.int32),
            pltpu.VMEM((_P,), jnp.int32),
            pltpu.VMEM((_G,), jnp.int32),
            pltpu.VMEM((_G,), jnp.int32),
            pltpu.VMEM((2, _G, _D // 2), jnp.int32),
            pltpu.VMEM((2, _C, _D), jnp.float32),
            pltpu.VMEM((_D,), jnp.float32),
            pltpu.VMEM((_D,), jnp.float32),
            pltpu.SemaphoreType.DMA,
            pltpu.SemaphoreType.DMA,
            pltpu.SemaphoreType.DMA,
            pltpu.SemaphoreType.DMA,
        ],
    )


def _pack_tc(table):
    n = table.shape[0]
    blk = 2000

    def body(t_ref, o_ref):
        u = lax.bitcast_convert_type(t_ref[...], jnp.int32)
        r = lax.shift_right_logical(
            u + 0x7FFF + lax.bitwise_and(
                lax.shift_right_logical(u, 16), 1), 16)
        ue = lax.slice(r, (0, 0), (blk, _D // 2))
        uo = lax.slice(r, (0, _D // 2), (blk, _D))
        o_ref[...] = lax.bitwise_or(ue, lax.shift_left(uo, 16))

    return pl.pallas_call(
        body,
        grid=(n // blk,),
        in_specs=[pl.BlockSpec((blk, _D), lambda i: (i, 0))],
        out_specs=pl.BlockSpec((blk, _D // 2), lambda i: (i, 0)),
        out_shape=jax.ShapeDtypeStruct((n, _D // 2), jnp.int32),
    )(table)


def _unpad_tc(out, L):
    B = out.shape[0]
    blk = 256

    def body(t_ref, o_ref):
        o_ref[...] = t_ref[:, :L, :]

    return pl.pallas_call(
        body,
        grid=(B // blk,),
        in_specs=[pl.BlockSpec((blk, out.shape[1], _D), lambda i: (i, 0, 0))],
        out_specs=pl.BlockSpec((blk, L, _D), lambda i: (i, 0, 0)),
        out_shape=jax.ShapeDtypeStruct((B, L, _D), jnp.float32),
    )(out)


def kernel(x, table, ln_gamma, ln_beta):
    B, L, F = x.shape
    R = B * L
    per_field = table.shape[0] // F
    pat = jnp.asarray(
        np.tile(np.arange(F, dtype=np.int32) * per_field, _P // F))
    tbl = _pack_tc(table)
    out = _build(B, L)(x.reshape(R * F), pat, tbl, ln_gamma, ln_beta)
    return _unpad_tc(out, L)


# speedup vs baseline: 1.0506x; 1.0506x over previous
import numpy as np
import jax
import jax.numpy as jnp
from jax import lax
from jax.experimental import pallas as pl
from jax.experimental.pallas import tpu as pltpu
from jax.experimental.pallas import tpu_sc as plsc

_F = 26
_D = 128
_NC = 2
_NS = 16
_NW = _NC * _NS
_C = 16
_G = _C * _F
_P = 208


_DNUMS = lax.GatherDimensionNumbers(
    offset_dims=(), collapsed_slice_dims=(0,), start_index_map=(0,))


def _perm(t, idx):
    return lax.gather(t, idx[:, None], _DNUMS, (1,),
                      mode=lax.GatherScatterMode.PROMISE_IN_BOUNDS)


def _lane_allsum(t):
    iota = lax.broadcasted_iota(jnp.int32, (16,), 0)
    for sh in (8, 4, 2, 1):
        t = t + _perm(t, iota ^ sh)
    return t


def _rsqrt_nr(x):
    i = lax.bitcast_convert_type(x, jnp.int32)
    y = lax.bitcast_convert_type(jnp.int32(0x5F3759DF) - (i >> 1), jnp.float32)
    for _ in range(3):
        y = y * (1.5 - 0.5 * x * y * y)
    return y


def _build(B, L):
    R = B * L
    rows_per_w = R // _NW
    iters = rows_per_w // _C
    words_x = rows_per_w * _F
    mesh = plsc.VectorSubcoreMesh(core_axis_name="c", subcore_axis_name="s")

    def body(x_ref, pat_ref, tbl_ref, gam_ref, bet_ref, out_ref,
             x_v, pat_v, idx0_v, idx1_v, rows_v, out_v, gam_v, bet_v,
             gsem0, gsem1, osem0, osem1):
        wid = lax.axis_index("s") * _NC + lax.axis_index("c")
        base_row = wid * rows_per_w
        pltpu.sync_copy(x_ref.at[pl.ds(base_row * _F, words_x)], x_v)
        pltpu.sync_copy(pat_ref, pat_v)
        pltpu.sync_copy(gam_ref, gam_v)
        pltpu.sync_copy(bet_ref, bet_v)

        gsems = (gsem0, gsem1)
        osems = (osem0, osem1)
        idxs = (idx0_v, idx1_v)


        def compute_idx(it, b):
            off = it * _G
            for j in range(_G // 16):
                idxs[b][pl.ds(j * 16, 16)] = (
                    x_v[pl.ds(off + j * 16, 16)]
                    + pat_v[pl.ds((j % (_P // 16)) * 16, 16)])

        def fire_gather(b):
            for q in range(_G // 104):
                pltpu.async_copy(
                    tbl_ref.at[idxs[b].at[pl.ds(q * 104, 104)]],
                    rows_v.at[b, pl.ds(q * 104, 104)], gsems[b])

        def wait_gather(b):
            pltpu.make_async_copy(
                tbl_ref.at[pl.ds(0, _G)], rows_v.at[b], gsems[b]).wait()

        def fire_out(it, b):
            for q in range(_C // 4):
                r0 = it * _C + q * 4
                bid = wid * (rows_per_w // L) + r0 // L
                l0 = r0 - (r0 // L) * L
                pltpu.async_copy(out_v.at[b, pl.ds(q * 4, 4)],
                                 out_ref.at[bid, pl.ds(l0, 4)], osems[b])

        def wait_out(b):
            pltpu.make_async_copy(
                out_ref.at[0, pl.ds(0, _C - 4)],
                out_v.at[b, pl.ds(0, _C - 4)], osems[b]).wait()
            pltpu.make_async_copy(
                out_ref.at[0, pl.ds(0, 4)],
                out_v.at[b, pl.ds(0, 4)], osems[b]).wait()

        compute_idx(0, 0)
        fire_gather(0)

        def step(i2, carry):
            for b in range(2):
                it = i2 * 2 + b
                nit = it + 1

                @pl.when(nit < iters)
                def _():
                    compute_idx(nit, 1 - b)
                    fire_gather(1 - b)

                wait_gather(b)

                @pl.when(it >= 2)
                def _():
                    wait_out(b)

                def row_body(r, c):
                    rbase = r * _F
                    sh16 = jnp.full((16,), 16, jnp.int32)
                    hi_mask = jnp.full((16,), -65536, jnp.int32)
                    acc = [None] * 8
                    for j in range(_F):
                        for k in range(4):
                            w = rows_v[b, rbase + j, pl.ds(k * 16, 16)]
                            ev = lax.bitcast_convert_type(
                                lax.shift_left(w, sh16), jnp.float32)
                            od = lax.bitcast_convert_type(
                                lax.bitwise_and(w, hi_mask), jnp.float32)
                            if j == 0:
                                acc[k] = ev
                                acc[4 + k] = od
                            else:
                                acc[k] = acc[k] + ev
                                acc[4 + k] = acc[4 + k] + od
                    t = acc[0]
                    for k in range(1, 8):
                        t = t + acc[k]
                    mv = _lane_allsum(t) * (1.0 / _D)
                    d = [a - mv for a in acc]
                    sq = d[0] * d[0]
                    for k in range(1, 8):
                        sq = sq + d[k] * d[k]
                    var = _lane_allsum(sq) * (1.0 / _D)
                    inv = _rsqrt_nr(var + 1e-5)
                    for k in range(8):
                        g = gam_v[pl.ds(k * 16, 16)] * inv
                        out_v[b, r, pl.ds(k * 16, 16)] = (
                            d[k] * g + bet_v[pl.ds(k * 16, 16)])
                    return c

                lax.fori_loop(0, _C, row_body, 0)
                fire_out(it, b)
            return carry

        lax.fori_loop(0, iters // 2, step, 0)
        wait_out(0)
        wait_out(1)

    return pl.kernel(
        body,
        out_type=jax.ShapeDtypeStruct((B, 24, _D), jnp.float32),
        mesh=mesh,
        compiler_params=pltpu.CompilerParams(use_tc_tiling_on_sc=False),
        scratch_types=[
            pltpu.VMEM((words_x,), jnp.int32),
            pltpu.VMEM((_P,), jnp.int32),
            pltpu.VMEM((_G,), jnp.int32),
            pltpu.VMEM((_G,), jnp.int32),
            pltpu.VMEM((2, _G, _D // 2), jnp.int32),
            pltpu.VMEM((2, _C, _D), jnp.float32),
            pltpu.VMEM((_D,), jnp.float32),
            pltpu.VMEM((_D,), jnp.float32),
            pltpu.SemaphoreType.DMA,
            pltpu.SemaphoreType.DMA,
            pltpu.SemaphoreType.DMA,
            pltpu.SemaphoreType.DMA,
        ],
    )


def _pack_tc(table):
    n = table.shape[0]
    blk = 2000

    def body(t_ref, o_ref):
        u = lax.bitcast_convert_type(t_ref[...], jnp.int32)
        r = lax.shift_right_logical(
            u + 0x7FFF + lax.bitwise_and(
                lax.shift_right_logical(u, 16), 1), 16)
        ue = lax.slice(r, (0, 0), (blk, _D // 2))
        uo = lax.slice(r, (0, _D // 2), (blk, _D))
        o_ref[...] = lax.bitwise_or(ue, lax.shift_left(uo, 16))

    return pl.pallas_call(
        body,
        grid=(n // blk,),
        in_specs=[pl.BlockSpec((blk, _D), lambda i: (i, 0))],
        out_specs=pl.BlockSpec((blk, _D // 2), lambda i: (i, 0)),
        out_shape=jax.ShapeDtypeStruct((n, _D // 2), jnp.int32),
    )(table)


def kernel(x, table, ln_gamma, ln_beta):
    B, L, F = x.shape
    R = B * L
    per_field = table.shape[0] // F
    pat = jnp.asarray(
        np.tile(np.arange(F, dtype=np.int32) * per_field, _P // F))
    tbl = _pack_tc(table)
    out = _build(B, L)(x.reshape(R * F), pat, tbl, ln_gamma, ln_beta)
    return out[:, :L, :]
